# baseline (speedup 1.0000x reference)
import jax
import jax.numpy as jnp
from jax import lax
from jax.experimental import pallas as pl
from jax.experimental.pallas import tpu as pltpu
from jax.experimental.pallas import tpu_sc as plsc

N = 10000
F_IN = 128
H = 16
E = 320000

NC = 2
NS = 16
EPT = E // (NC * NS)
K = 2000
NCHUNK = EPT // K
NP = 10240
NPS = NP // NS


def _sc_deg_body(ei_hbm, w_hbm, degp_hbm, col_v0, col_v1, w_v0, w_v1,
                 deg_sh, sem0, sem1):
    cid = lax.axis_index("c")
    sid = lax.axis_index("s")
    ebase = (cid * NS + sid) * EPT
    nsl = pl.ds(sid * NPS, NPS)

    zv = jnp.zeros((16,), jnp.float32)
    def zfill(g, c):
        w_v0[pl.ds(g * 16, 16)] = zv
        return c
    lax.fori_loop(0, NPS // 16, zfill, 0)
    pltpu.sync_copy(w_v0.at[pl.ds(0, NPS)], deg_sh.at[nsl])
    plsc.subcore_barrier()

    sems = (sem0, sem1)
    col_b = (col_v0, col_v1)
    w_b = (w_v0, w_v1)
    pltpu.sync_copy(ei_hbm.at[1, pl.ds(ebase, K)], col_v0)
    pltpu.sync_copy(w_hbm.at[pl.ds(ebase, K)], w_v0)
    loads = [None, None]
    for i in range(NCHUNK):
        b = i % 2
        nb = 1 - b
        if i + 1 < NCHUNK:
            loads[nb] = (
                pltpu.async_copy(ei_hbm.at[1, pl.ds(ebase + (i + 1) * K, K)],
                                 col_b[nb], sems[nb]),
                pltpu.async_copy(w_hbm.at[pl.ds(ebase + (i + 1) * K, K)],
                                 w_b[nb], sems[nb]),
            )
        pltpu.sync_copy(w_b[b], deg_sh.at[col_b[b]], add=True)
        if i + 1 < NCHUNK:
            loads[nb][0].wait()
            loads[nb][1].wait()

    plsc.subcore_barrier()
    pltpu.sync_copy(deg_sh.at[nsl], degp_hbm.at[cid, nsl])


def _sc_prop_body(ei_hbm, w_hbm, g_hbm, accp_hbm,
                  row_v0, row_v1, col_v0, col_v1, w_v0, w_v1,
                  msg_v0, msg_v1, acc_sh, gsem0, gsem1, ssem0, ssem1):
    cid = lax.axis_index("c")
    sid = lax.axis_index("s")
    ebase = (cid * NS + sid) * EPT
    nsl = pl.ds(sid * NPS, NPS)

    zv = jnp.zeros((16,), jnp.float32)
    def zfill(e, c):
        msg_v0[e] = zv
        return c
    lax.fori_loop(0, NPS, zfill, 0)
    pltpu.sync_copy(msg_v0.at[pl.ds(0, NPS)], acc_sh.at[nsl])

    gsems = (gsem0, gsem1)
    row_b = (row_v0, row_v1)
    col_b = (col_v0, col_v1)
    w_b = (w_v0, w_v1)
    msg_b = (msg_v0, msg_v1)

    def load_idx(i, b):
        pltpu.sync_copy(ei_hbm.at[0, pl.ds(ebase + i * K, K)], row_b[b])
        pltpu.sync_copy(ei_hbm.at[1, pl.ds(ebase + i * K, K)], col_b[b])
        pltpu.sync_copy(w_hbm.at[pl.ds(ebase + i * K, K)], w_b[b])

    ssems = (ssem0, ssem1)
    load_idx(0, 0)
    gathers = [pltpu.async_copy(g_hbm.at[row_v0], msg_v0, gsems[0]),
               None]
    plsc.subcore_barrier()
    scats = [None, None]
    for i in range(NCHUNK):
        b = i % 2
        nb = 1 - b
        if i + 1 < NCHUNK:
            if scats[nb] is not None:
                scats[nb].wait()
            load_idx(i + 1, nb)
            gathers[nb] = pltpu.async_copy(g_hbm.at[row_b[nb]],
                                           msg_b[nb], gsems[nb])
        gathers[b].wait()

        mv = msg_b[b]
        wv_ref = w_b[b]

        @plsc.parallel_loop(0, K, step=16, unroll=4)
        def _scale(e0):
            wv = wv_ref[pl.ds(e0, 16)]
            for k in range(16):
                e = e0 + k
                mv[e] = mv[e] * wv[k]
        scats[b] = pltpu.async_copy(mv, acc_sh.at[col_b[b]], ssems[b],
                                    add=True)

    for sc in scats:
        if sc is not None:
            sc.wait()
    plsc.subcore_barrier()
    pltpu.sync_copy(acc_sh.at[nsl], accp_hbm.at[cid, nsl])


_SC_MESH = plsc.VectorSubcoreMesh(
    core_axis_name="c", subcore_axis_name="s", num_cores=NC, num_subcores=NS)

_deg_call = pl.kernel(
    _sc_deg_body,
    out_type=jax.ShapeDtypeStruct((NC, NP), jnp.float32),
    mesh=_SC_MESH,
    compiler_params=pltpu.CompilerParams(use_tc_tiling_on_sc=False,
                                         needs_layout_passes=False),
    scratch_types=[
        pltpu.VMEM((K,), jnp.int32),
        pltpu.VMEM((K,), jnp.int32),
        pltpu.VMEM((K,), jnp.float32),
        pltpu.VMEM((K,), jnp.float32),
        pltpu.VMEM_SHARED((NP,), jnp.float32),
        pltpu.SemaphoreType.DMA,
        pltpu.SemaphoreType.DMA,
    ],
)

_prop_call = pl.kernel(
    _sc_prop_body,
    out_type=jax.ShapeDtypeStruct((NC, NP, H), jnp.float32),
    mesh=_SC_MESH,
    compiler_params=pltpu.CompilerParams(use_tc_tiling_on_sc=False,
                                         needs_layout_passes=False),
    scratch_types=[
        pltpu.VMEM((K,), jnp.int32),
        pltpu.VMEM((K,), jnp.int32),
        pltpu.VMEM((K,), jnp.int32),
        pltpu.VMEM((K,), jnp.int32),
        pltpu.VMEM((K,), jnp.float32),
        pltpu.VMEM((K,), jnp.float32),
        pltpu.VMEM((K, H), jnp.float32),
        pltpu.VMEM((K, H), jnp.float32),
        pltpu.VMEM_SHARED((NP, H), jnp.float32),
        pltpu.SemaphoreType.DMA,
        pltpu.SemaphoreType.DMA,
        pltpu.SemaphoreType.DMA,
        pltpu.SemaphoreType.DMA,
    ],
)


def _dense_a(x_ref, fm_ref, w1_ref, fm_out, h1_out):
    fm = jax.nn.sigmoid(fm_ref[...])
    fm_out[...] = fm
    xm = x_ref[...] * fm
    h1_out[...] = jnp.dot(xm, w1_ref[...], preferred_element_type=jnp.float32)


def _dense_c(degp_ref, h1_ref, dis_out, dis2_out, g1_out):
    deg = (degp_ref[0, :N] + degp_ref[1, :N] + 1.0)[:, None]
    dis = jax.lax.rsqrt(deg)
    disb = jnp.broadcast_to(dis, (N, H))
    dis_out[...] = disb
    dis2_out[...] = disb * disb
    g1_out[...] = disb * h1_ref[...]


def _dense_e(acc1_ref, dis_ref, dis2_ref, h1_ref, b1_ref, w2_ref,
             h2_out, g2_out):
    acc = acc1_ref[0, :N] + acc1_ref[1, :N]
    out1 = jax.nn.relu(dis_ref[...] * acc + dis2_ref[...] * h1_ref[...]
                       + b1_ref[...])
    h2 = jnp.dot(out1, w2_ref[...], preferred_element_type=jnp.float32)
    h2_out[...] = h2
    g2_out[...] = dis_ref[...] * h2


def _dense_g(acc2_ref, dis_ref, dis2_ref, h2_ref, b2_ref, out_ref):
    acc = acc2_ref[0, :N] + acc2_ref[1, :N]
    o = dis_ref[...] * acc + dis2_ref[...] * h2_ref[...] + b2_ref[...]
    mask = jax.lax.broadcasted_iota(jnp.int32, o.shape, 1) < 7
    neg = jnp.full_like(o, -jnp.inf)
    om = jnp.where(mask, o, neg)
    m = jnp.max(om, axis=1, keepdims=True)
    ex = jnp.where(mask, jnp.exp(o - m), jnp.zeros_like(o))
    lse = jnp.log(jnp.sum(ex, axis=1, keepdims=True))
    out_ref[...] = (o - m - lse)[:, :7]


def kernel(x, edge_index, edge_weight, feat_mask, W1, b1, W2, b2):
    ei = edge_index.astype(jnp.int32)
    w = edge_weight.astype(jnp.float32)

    fm, h1 = pl.pallas_call(
        _dense_a,
        out_shape=[jax.ShapeDtypeStruct((N, F_IN), jnp.float32),
                   jax.ShapeDtypeStruct((N, H), jnp.float32)],
    )(x, feat_mask, W1)

    degp = _deg_call(ei, w)

    dis, dis2, g1 = pl.pallas_call(
        _dense_c,
        out_shape=[jax.ShapeDtypeStruct((N, H), jnp.float32),
                   jax.ShapeDtypeStruct((N, H), jnp.float32),
                   jax.ShapeDtypeStruct((N, H), jnp.float32)],
    )(degp, h1)

    acc1 = _prop_call(ei, w, g1)

    W2p = jnp.zeros((H, H), jnp.float32).at[:, :W2.shape[1]].set(W2)
    b1r = b1[None, :]
    b2p = jnp.zeros((1, H), jnp.float32).at[0, :b2.shape[0]].set(b2)

    h2, g2 = pl.pallas_call(
        _dense_e,
        out_shape=[jax.ShapeDtypeStruct((N, H), jnp.float32),
                   jax.ShapeDtypeStruct((N, H), jnp.float32)],
    )(acc1, dis, dis2, h1, b1r, W2p)

    acc2 = _prop_call(ei, w, g2)

    outp = pl.pallas_call(
        _dense_g,
        out_shape=jax.ShapeDtypeStruct((N, 7), jnp.float32),
    )(acc2, dis, dis2, h2, b2p)

    return outp, fm


# speedup vs baseline: 1.0460x; 1.0460x over previous
import jax
import jax.numpy as jnp
from jax import lax
from jax.experimental import pallas as pl
from jax.experimental.pallas import tpu as pltpu
from jax.experimental.pallas import tpu_sc as plsc

N = 10000
F_IN = 128
H = 16
E = 320000

NC = 2
NS = 16
EPT = E // (NC * NS)
K = 2000
NCHUNK = EPT // K
NP = 10240
NPS = NP // NS


def _sc_deg_body(ei_hbm, w_hbm, degp_hbm, col_v0, col_v1, w_v0, w_v1,
                 deg_sh, sem0, sem1):
    cid = lax.axis_index("c")
    sid = lax.axis_index("s")
    ebase = (cid * NS + sid) * EPT
    nsl = pl.ds(sid * NPS, NPS)

    zv = jnp.zeros((16,), jnp.float32)
    def zfill(g, c):
        w_v0[pl.ds(g * 16, 16)] = zv
        return c
    lax.fori_loop(0, NPS // 16, zfill, 0)
    pltpu.sync_copy(w_v0.at[pl.ds(0, NPS)], deg_sh.at[nsl])
    plsc.subcore_barrier()

    sems = (sem0, sem1)
    col_b = (col_v0, col_v1)
    w_b = (w_v0, w_v1)
    pltpu.sync_copy(ei_hbm.at[1, pl.ds(ebase, K)], col_v0)
    pltpu.sync_copy(w_hbm.at[pl.ds(ebase, K)], w_v0)
    loads = [None, None]
    for i in range(NCHUNK):
        b = i % 2
        nb = 1 - b
        if i + 1 < NCHUNK:
            loads[nb] = (
                pltpu.async_copy(ei_hbm.at[1, pl.ds(ebase + (i + 1) * K, K)],
                                 col_b[nb], sems[nb]),
                pltpu.async_copy(w_hbm.at[pl.ds(ebase + (i + 1) * K, K)],
                                 w_b[nb], sems[nb]),
            )
        pltpu.sync_copy(w_b[b], deg_sh.at[col_b[b]], add=True)
        if i + 1 < NCHUNK:
            loads[nb][0].wait()
            loads[nb][1].wait()

    plsc.subcore_barrier()
    pltpu.sync_copy(deg_sh.at[nsl], degp_hbm.at[cid, nsl])


def _sc_prop_body(ei_hbm, w_hbm, g_hbm, accp_hbm,
                  row_v0, row_v1, col_v0, col_v1, w_v0, w_v1,
                  msg_v0, msg_v1, acc_sh, gsem0, gsem1, ssem0, ssem1):
    cid = lax.axis_index("c")
    sid = lax.axis_index("s")
    ebase = (cid * NS + sid) * EPT
    nsl = pl.ds(sid * NPS, NPS)

    zv = jnp.zeros((16,), jnp.float32)
    def zfill(e, c):
        msg_v0[e] = zv
        return c
    lax.fori_loop(0, NPS, zfill, 0)
    pltpu.sync_copy(msg_v0.at[pl.ds(0, NPS)], acc_sh.at[nsl])

    gsems = (gsem0, gsem1)
    row_b = (row_v0, row_v1)
    col_b = (col_v0, col_v1)
    w_b = (w_v0, w_v1)
    msg_b = (msg_v0, msg_v1)

    def load_idx(i, b):
        pltpu.sync_copy(ei_hbm.at[0, pl.ds(ebase + i * K, K)], row_b[b])
        pltpu.sync_copy(ei_hbm.at[1, pl.ds(ebase + i * K, K)], col_b[b])
        pltpu.sync_copy(w_hbm.at[pl.ds(ebase + i * K, K)], w_b[b])

    ssems = (ssem0, ssem1)
    load_idx(0, 0)
    gathers = [pltpu.async_copy(g_hbm.at[row_v0], msg_v0, gsems[0]),
               None]
    plsc.subcore_barrier()
    scats = [None, None]
    for i in range(NCHUNK):
        b = i % 2
        nb = 1 - b
        if i + 1 < NCHUNK:
            if scats[nb] is not None:
                scats[nb].wait()
            load_idx(i + 1, nb)
            gathers[nb] = pltpu.async_copy(g_hbm.at[row_b[nb]],
                                           msg_b[nb], gsems[nb])
        gathers[b].wait()

        mv = msg_b[b]
        wv_ref = w_b[b]

        @plsc.parallel_loop(0, K, step=16, unroll=2)
        def _scale(e0):
            wv = wv_ref[pl.ds(e0, 16)]
            for k in range(16):
                e = e0 + k
                mv[e] = mv[e] * wv[k]
        scats[b] = pltpu.async_copy(mv, acc_sh.at[col_b[b]], ssems[b],
                                    add=True)

    for sc in scats:
        if sc is not None:
            sc.wait()
    plsc.subcore_barrier()
    pltpu.sync_copy(acc_sh.at[nsl], accp_hbm.at[cid, nsl])


_SC_MESH = plsc.VectorSubcoreMesh(
    core_axis_name="c", subcore_axis_name="s", num_cores=NC, num_subcores=NS)

_deg_call = pl.kernel(
    _sc_deg_body,
    out_type=jax.ShapeDtypeStruct((NC, NP), jnp.float32),
    mesh=_SC_MESH,
    compiler_params=pltpu.CompilerParams(use_tc_tiling_on_sc=False,
                                         needs_layout_passes=False),
    scratch_types=[
        pltpu.VMEM((K,), jnp.int32),
        pltpu.VMEM((K,), jnp.int32),
        pltpu.VMEM((K,), jnp.float32),
        pltpu.VMEM((K,), jnp.float32),
        pltpu.VMEM_SHARED((NP,), jnp.float32),
        pltpu.SemaphoreType.DMA,
        pltpu.SemaphoreType.DMA,
    ],
)

_prop_call = pl.kernel(
    _sc_prop_body,
    out_type=jax.ShapeDtypeStruct((NC, NP, H), jnp.float32),
    mesh=_SC_MESH,
    compiler_params=pltpu.CompilerParams(use_tc_tiling_on_sc=False,
                                         needs_layout_passes=False),
    scratch_types=[
        pltpu.VMEM((K,), jnp.int32),
        pltpu.VMEM((K,), jnp.int32),
        pltpu.VMEM((K,), jnp.int32),
        pltpu.VMEM((K,), jnp.int32),
        pltpu.VMEM((K,), jnp.float32),
        pltpu.VMEM((K,), jnp.float32),
        pltpu.VMEM((K, H), jnp.float32),
        pltpu.VMEM((K, H), jnp.float32),
        pltpu.VMEM_SHARED((NP, H), jnp.float32),
        pltpu.SemaphoreType.DMA,
        pltpu.SemaphoreType.DMA,
        pltpu.SemaphoreType.DMA,
        pltpu.SemaphoreType.DMA,
    ],
)


def _dense_a(x_ref, fm_ref, w1_ref, fm_out, h1_out):
    fm = jax.nn.sigmoid(fm_ref[...])
    fm_out[...] = fm
    xm = x_ref[...] * fm
    h1_out[...] = jnp.dot(xm, w1_ref[...], preferred_element_type=jnp.float32)


def _dense_c(degp_ref, h1_ref, dis_out, dis2_out, g1_out):
    deg = (degp_ref[0, :N] + degp_ref[1, :N] + 1.0)[:, None]
    dis = jax.lax.rsqrt(deg)
    disb = jnp.broadcast_to(dis, (N, H))
    dis_out[...] = disb
    dis2_out[...] = disb * disb
    g1_out[...] = disb * h1_ref[...]


def _dense_e(acc1_ref, dis_ref, dis2_ref, h1_ref, b1_ref, w2_ref,
             h2_out, g2_out):
    acc = acc1_ref[0, :N] + acc1_ref[1, :N]
    out1 = jax.nn.relu(dis_ref[...] * acc + dis2_ref[...] * h1_ref[...]
                       + b1_ref[...])
    h2 = jnp.dot(out1, w2_ref[...], preferred_element_type=jnp.float32)
    h2_out[...] = h2
    g2_out[...] = dis_ref[...] * h2


def _dense_g(acc2_ref, dis_ref, dis2_ref, h2_ref, b2_ref, out_ref):
    acc = acc2_ref[0, :N] + acc2_ref[1, :N]
    o = dis_ref[...] * acc + dis2_ref[...] * h2_ref[...] + b2_ref[...]
    mask = jax.lax.broadcasted_iota(jnp.int32, o.shape, 1) < 7
    neg = jnp.full_like(o, -jnp.inf)
    om = jnp.where(mask, o, neg)
    m = jnp.max(om, axis=1, keepdims=True)
    ex = jnp.where(mask, jnp.exp(o - m), jnp.zeros_like(o))
    lse = jnp.log(jnp.sum(ex, axis=1, keepdims=True))
    out_ref[...] = (o - m - lse)[:, :7]


def kernel(x, edge_index, edge_weight, feat_mask, W1, b1, W2, b2):
    ei = edge_index.astype(jnp.int32)
    w = edge_weight.astype(jnp.float32)

    fm, h1 = pl.pallas_call(
        _dense_a,
        out_shape=[jax.ShapeDtypeStruct((N, F_IN), jnp.float32),
                   jax.ShapeDtypeStruct((N, H), jnp.float32)],
    )(x, feat_mask, W1)

    degp = _deg_call(ei, w)

    dis, dis2, g1 = pl.pallas_call(
        _dense_c,
        out_shape=[jax.ShapeDtypeStruct((N, H), jnp.float32),
                   jax.ShapeDtypeStruct((N, H), jnp.float32),
                   jax.ShapeDtypeStruct((N, H), jnp.float32)],
    )(degp, h1)

    acc1 = _prop_call(ei, w, g1)

    W2p = jnp.zeros((H, H), jnp.float32).at[:, :W2.shape[1]].set(W2)
    b1r = b1[None, :]
    b2p = jnp.zeros((1, H), jnp.float32).at[0, :b2.shape[0]].set(b2)

    h2, g2 = pl.pallas_call(
        _dense_e,
        out_shape=[jax.ShapeDtypeStruct((N, H), jnp.float32),
                   jax.ShapeDtypeStruct((N, H), jnp.float32)],
    )(acc1, dis, dis2, h1, b1r, W2p)

    acc2 = _prop_call(ei, w, g2)

    outp = pl.pallas_call(
        _dense_g,
        out_shape=jax.ShapeDtypeStruct((N, 7), jnp.float32),
    )(acc2, dis, dis2, h2, b2p)

    return outp, fm
